# baseline (speedup 1.0000x reference)
import functools

import jax
import jax.numpy as jnp
from jax import lax
from jax.experimental import pallas as pl
from jax.experimental.pallas import tpu as pltpu
from jax.experimental.pallas import tpu_sc as plsc

B = 2048
E = 1024
NB = 8
NCLASS = 1000
TB = 128
NT = 24
PAD = NT * TB
BT = 256
CS = 128
NCP = 1024

_f32 = jnp.float32


def _sc_perm(table, idx_in, idx_out, out_rows):
    V, D = table.shape
    Bc = (idx_in if idx_in is not None else idx_out).shape[0]
    info = plsc.get_sparse_core_info()
    NC, NS = info.num_cores, info.num_subcores
    NW = NC * NS
    npw = Bc // NW
    NCK = 2
    cn = npw // NCK
    assert Bc % (8 * NW) == 0 and D % 128 == 0 and cn % 8 == 0
    mesh = plsc.VectorSubcoreMesh(core_axis_name="c", subcore_axis_name="s")

    scratch = []
    if idx_in is not None:
        scratch += [pltpu.VMEM((cn,), jnp.int32)] * NCK
    if idx_out is not None:
        scratch += [pltpu.VMEM((cn,), jnp.int32)] * NCK
    scratch += [pltpu.VMEM((cn, D), _f32)] * NCK
    scratch += [pltpu.SemaphoreType.DMA] * (2 * NCK)

    @functools.partial(
        pl.kernel, mesh=mesh,
        out_type=jax.ShapeDtypeStruct((out_rows, D), _f32),
        scratch_types=scratch,
    )
    def perm_k(*refs):
        pos = 0
        table_hbm = refs[pos]; pos += 1
        ii_hbm = io_hbm = None
        if idx_in is not None:
            ii_hbm = refs[pos]; pos += 1
        if idx_out is not None:
            io_hbm = refs[pos]; pos += 1
        out_hbm = refs[pos]; pos += 1
        ii_v = io_v = None
        if idx_in is not None:
            ii_v = refs[pos:pos + NCK]; pos += NCK
        if idx_out is not None:
            io_v = refs[pos:pos + NCK]; pos += NCK
        rows_v = refs[pos:pos + NCK]; pos += NCK
        sem_g = refs[pos:pos + NCK]; pos += NCK
        sem_s = refs[pos:pos + NCK]

        wid = lax.axis_index("s") * NC + lax.axis_index("c")
        base = wid * npw

        gathers = []
        for k in range(NCK):
            if idx_in is not None:
                pltpu.sync_copy(ii_hbm.at[pl.ds(base + k * cn, cn)], ii_v[k])
            if idx_out is not None:
                pltpu.sync_copy(io_hbm.at[pl.ds(base + k * cn, cn)], io_v[k])
        for k in range(NCK):
            if idx_in is not None:
                g = pltpu.async_copy(table_hbm.at[ii_v[k]], rows_v[k],
                                     sem_g[k])
            else:
                g = pltpu.async_copy(table_hbm.at[pl.ds(base + k * cn, cn)],
                                     rows_v[k], sem_g[k])
            gathers.append(g)
        scatters = []
        for k in range(NCK):
            gathers[k].wait()
            if idx_out is not None:
                s = pltpu.async_copy(rows_v[k], out_hbm.at[io_v[k]], sem_s[k])
            else:
                s = pltpu.async_copy(rows_v[k],
                                     out_hbm.at[pl.ds(base + k * cn, cn)],
                                     sem_s[k])
            scatters.append(s)
        for s in scatters:
            s.wait()

    args = [table]
    if idx_in is not None:
        args.append(idx_in)
    if idx_out is not None:
        args.append(idx_out)
    return perm_k(*args)


def _embed_head(x, emb_W, emb_b, at_W1, at_b1, at_W2, at_b2):
    def body(x_ref, eW, eb, w1, b1, w2, b2, state_ref, lg_ref):
        st = jnp.dot(x_ref[...], eW[...], preferred_element_type=_f32) + eb[...]
        state_ref[...] = st
        h = jnp.maximum(jnp.dot(st, w1[0], preferred_element_type=_f32) + b1[0], 0.0)
        lg_ref[...] = jnp.dot(h, w2[0], preferred_element_type=_f32) + b2[0]

    return pl.pallas_call(
        body,
        grid=(B // BT,),
        in_specs=[
            pl.BlockSpec((BT, E), lambda i: (i, 0)),
            pl.BlockSpec((E, E), lambda i: (0, 0)),
            pl.BlockSpec((1, E), lambda i: (0, 0)),
            pl.BlockSpec((1, E, E), lambda i: (0, 0, 0)),
            pl.BlockSpec((1, 1, E), lambda i: (0, 0, 0)),
            pl.BlockSpec((1, E, NB), lambda i: (0, 0, 0)),
            pl.BlockSpec((1, 1, NB), lambda i: (0, 0, 0)),
        ],
        out_specs=[
            pl.BlockSpec((BT, E), lambda i: (i, 0)),
            pl.BlockSpec((BT, NB), lambda i: (i, 0)),
        ],
        out_shape=[
            jax.ShapeDtypeStruct((B, E), _f32),
            jax.ShapeDtypeStruct((B, NB), _f32),
        ],
        compiler_params=pltpu.CompilerParams(
            dimension_semantics=("arbitrary",)),
    )(x, emb_W, emb_b, at_W1, at_b1, at_W2, at_b2)


def _jump_with_addr(xs, te, nu, W1, b1, W2, b2, A1, Ab1, A2p, Ab2p):
    def body(te_r, nu_r, x_ref, w1, b1_, w2, b2_, a1, ab1, a2, ab2,
             ns_ref, lg_ref):
        t = pl.program_id(0)

        @pl.when(t < nu_r[0])
        def _():
            x = x_ref[...]
            norm = jnp.sqrt(jnp.sum(x * x, axis=1, keepdims=True))
            h1 = jnp.maximum(jnp.dot(x, w1[0], preferred_element_type=_f32) + b1_[0], 0.0)
            h2 = jnp.maximum(jnp.dot(h1, w2[0], preferred_element_type=_f32) + b2_[0], 0.0)
            ns = h2 / (norm + 1e-6)
            ns_ref[...] = ns
            av = jnp.maximum(jnp.dot(ns, a1[0], preferred_element_type=_f32) + ab1[0], 0.0)
            z = jnp.dot(av, a2[0], preferred_element_type=_f32) + ab2[0]
            lg_ref[...] = jnp.concatenate(
                [z, jnp.full((TB, 128 - NB), -1e9, _f32)], axis=1)

    def xmap(t, te_r, nu_r):
        return (jnp.minimum(t, nu_r[0] - 1), 0)

    def wmap(t, te_r, nu_r):
        return (te_r[t], 0, 0)

    grid_spec = pltpu.PrefetchScalarGridSpec(
        num_scalar_prefetch=2,
        grid=(NT,),
        in_specs=[
            pl.BlockSpec((TB, E), xmap),
            pl.BlockSpec((1, E, E), wmap),
            pl.BlockSpec((1, 1, E), wmap),
            pl.BlockSpec((1, E, E), wmap),
            pl.BlockSpec((1, 1, E), wmap),
            pl.BlockSpec((1, E, E), wmap),
            pl.BlockSpec((1, 1, E), wmap),
            pl.BlockSpec((1, E, NB), wmap),
            pl.BlockSpec((1, 1, NB), wmap),
        ],
        out_specs=[
            pl.BlockSpec((TB, E), xmap),
            pl.BlockSpec((TB, 128), xmap),
        ],
    )
    return pl.pallas_call(
        body,
        grid_spec=grid_spec,
        out_shape=[
            jax.ShapeDtypeStruct((PAD, E), _f32),
            jax.ShapeDtypeStruct((PAD, 128), _f32),
        ],
        compiler_params=pltpu.CompilerParams(
            dimension_semantics=("arbitrary",)),
    )(te, nu, xs, W1, b1, W2, b2, A1, Ab1, A2p, Ab2p)


def _jump_out(xs, te, nu, W1, b1, W2, b2, oW1, ob1, oW2, ob2):
    def body(te_r, nu_r, x_ref, w1, b1_, w2, b2_, ow1, ob1_, ow2, ob2_,
             o_ref):
        t = pl.program_id(0)

        @pl.when(t < nu_r[0])
        def _():
            x = x_ref[...]
            norm = jnp.sqrt(jnp.sum(x * x, axis=1, keepdims=True))
            h1 = jnp.maximum(jnp.dot(x, w1[0], preferred_element_type=_f32) + b1_[0], 0.0)
            h2 = jnp.maximum(jnp.dot(h1, w2[0], preferred_element_type=_f32) + b2_[0], 0.0)
            ns = h2 / (norm + 1e-6)
            ho = jnp.maximum(jnp.dot(ns, ow1[...], preferred_element_type=_f32) + ob1_[...], 0.0)
            o = jnp.dot(ho, ow2[...], preferred_element_type=_f32) + ob2_[...]
            o_ref[...] = jnp.concatenate(
                [o, jnp.zeros((TB, NCP - NCLASS), _f32)], axis=1)

    def xmap(t, te_r, nu_r):
        return (jnp.minimum(t, nu_r[0] - 1), 0)

    def wmap(t, te_r, nu_r):
        return (te_r[t], 0, 0)

    def cmap2(t, te_r, nu_r):
        return (0, 0)

    grid_spec = pltpu.PrefetchScalarGridSpec(
        num_scalar_prefetch=2,
        grid=(NT,),
        in_specs=[
            pl.BlockSpec((TB, E), xmap),
            pl.BlockSpec((1, E, E), wmap),
            pl.BlockSpec((1, 1, E), wmap),
            pl.BlockSpec((1, E, E), wmap),
            pl.BlockSpec((1, 1, E), wmap),
            pl.BlockSpec((E, E), cmap2),
            pl.BlockSpec((1, E), cmap2),
            pl.BlockSpec((E, NCLASS), cmap2),
            pl.BlockSpec((1, NCLASS), cmap2),
        ],
        out_specs=pl.BlockSpec((TB, NCP), xmap),
    )
    return pl.pallas_call(
        body,
        grid_spec=grid_spec,
        out_shape=jax.ShapeDtypeStruct((PAD, NCP), _f32),
        compiler_params=pltpu.CompilerParams(
            dimension_semantics=("arbitrary",)),
    )(te, nu, xs, W1, b1, W2, b2, oW1, ob1, oW2, ob2)


def _gumbel(i):
    with jax.ensure_compile_time_eval():
        key = jax.random.fold_in(jax.random.key(42), i)
        u = jax.random.uniform(key, (B, 1, NB), minval=1e-10, maxval=1.0)
        return (-jnp.log(-jnp.log(u))).reshape(B, NB)


def _route_build(lg, g):
    W = lg.shape[1]

    def body(lg_ref, g_ref, ppos_ref, te_ref, nu_ref):
        z = lg_ref[:, :NB] + g_ref[...]
        m = jnp.max(z, axis=1, keepdims=True)
        io8 = lax.broadcasted_iota(jnp.int32, (B, NB), 1)
        eidx = jnp.min(jnp.where(z == m, io8, NB), axis=1, keepdims=True)
        oh = (eidx == io8).astype(_f32)
        r = lax.broadcasted_iota(jnp.int32, (CS, CS), 0)
        c = lax.broadcasted_iota(jnp.int32, (CS, CS), 1)
        ltri = (c <= r).astype(_f32)
        parts = []
        carry = jnp.zeros((1, NB), _f32)
        for gi in range(B // CS):
            cs = jnp.dot(ltri, oh[gi * CS:(gi + 1) * CS, :],
                         preferred_element_type=_f32) + carry
            parts.append(cs)
            carry = cs[CS - 1:CS, :]
        csum = jnp.concatenate(parts, axis=0)
        rank = jnp.sum(oh * csum, axis=1, keepdims=True) - 1.0
        counts = carry
        tiles = jnp.floor((counts + (TB - 1)) * (1.0 / TB))
        u8r = lax.broadcasted_iota(jnp.int32, (NB, NB), 0)
        u8c = lax.broadcasted_iota(jnp.int32, (NB, NB), 1)
        utri = (u8r <= u8c).astype(_f32)
        cumt = jnp.dot(tiles, utri, preferred_element_type=_f32)
        nused = cumt[0:1, NB - 1:NB]
        tile_start = (cumt - tiles) * float(TB)
        ppos = jnp.sum(oh * tile_start, axis=1, keepdims=True) + rank
        ppos_ref[...] = ppos.astype(jnp.int32)
        tio = lax.broadcasted_iota(jnp.int32, (NT, 1), 0).astype(_f32)
        tcl = jnp.minimum(tio, nused - 1.0)
        te_ref[...] = jnp.sum((cumt <= tcl).astype(jnp.int32), axis=1,
                              keepdims=True)
        nu_ref[...] = nused.astype(jnp.int32)

    ppos, te, nu = pl.pallas_call(
        body,
        in_specs=[
            pl.BlockSpec((B, W), lambda: (0, 0)),
            pl.BlockSpec((B, NB), lambda: (0, 0)),
        ],
        out_specs=[
            pl.BlockSpec((B, 1), lambda: (0, 0)),
            pl.BlockSpec((NT, 1), lambda: (0, 0)),
            pl.BlockSpec((1, 1), lambda: (0, 0)),
        ],
        out_shape=[
            jax.ShapeDtypeStruct((B, 1), jnp.int32),
            jax.ShapeDtypeStruct((NT, 1), jnp.int32),
            jax.ShapeDtypeStruct((1, 1), jnp.int32),
        ],
    )(lg, g)
    return ppos.reshape(B), te.reshape(NT), nu.reshape(1)


def kernel(x, emb_W, emb_b, st_W1, st_b1, st_W2, st_b2,
           at_W1, at_b1, at_W2, at_b2, out_W1, out_b1, out_W2, out_b2):
    emb_b2 = emb_b.reshape(1, E)
    st_b1r = st_b1.reshape(NB, 1, E)
    st_b2r = st_b2.reshape(NB, 1, E)
    at_b1r = at_b1.reshape(NB, 1, E)
    at_b2r = at_b2.reshape(NB, 1, NB)

    at_b2r = at_b2.reshape(NB, 1, NB)

    state, lg0 = _embed_head(x, emb_W, emb_b2, at_W1, at_b1r, at_W2, at_b2r)

    ppos1, te1, nu1 = _route_build(lg0, _gumbel(0))
    xs1 = _sc_perm(state, None, ppos1, PAD)
    ns1, lgp1 = _jump_with_addr(xs1, te1, nu1, st_W1, st_b1r, st_W2, st_b2r,
                                at_W1, at_b1r, at_W2, at_b2r)

    lg1 = _sc_perm(lgp1, ppos1, None, B)
    ppos2, te2, nu2 = _route_build(lg1, _gumbel(1))
    xs2 = _sc_perm(ns1, ppos1, ppos2, PAD)
    o_sorted = _jump_out(xs2, te2, nu2, st_W1, st_b1r, st_W2, st_b2r,
                         out_W1, out_b1.reshape(1, E),
                         out_W2, out_b2.reshape(1, NCLASS))

    o = _sc_perm(o_sorted, ppos2, None, B)
    return o[:, :NCLASS]


# speedup vs baseline: 1.0076x; 1.0076x over previous
import functools

import jax
import jax.numpy as jnp
from jax import lax
from jax.experimental import pallas as pl
from jax.experimental.pallas import tpu as pltpu
from jax.experimental.pallas import tpu_sc as plsc

B = 2048
E = 1024
NB = 8
NCLASS = 1000
TB = 128
NT = 24
PAD = NT * TB
BT = 256
CS = 128
NCP = 1024

_f32 = jnp.float32


def _sc_perm(table, idx_in, idx_out, out_rows):
    V, D = table.shape
    Bc = (idx_in if idx_in is not None else idx_out).shape[0]
    info = plsc.get_sparse_core_info()
    NC, NS = info.num_cores, info.num_subcores
    NW = NC * NS
    npw = Bc // NW
    NCK = 1
    cn = npw // NCK
    assert Bc % (8 * NW) == 0 and D % 128 == 0 and cn % 8 == 0
    mesh = plsc.VectorSubcoreMesh(core_axis_name="c", subcore_axis_name="s")

    scratch = []
    if idx_in is not None:
        scratch += [pltpu.VMEM((cn,), jnp.int32)] * NCK
    if idx_out is not None:
        scratch += [pltpu.VMEM((cn,), jnp.int32)] * NCK
    scratch += [pltpu.VMEM((cn, D), _f32)] * NCK
    scratch += [pltpu.SemaphoreType.DMA] * (2 * NCK)

    @functools.partial(
        pl.kernel, mesh=mesh,
        out_type=jax.ShapeDtypeStruct((out_rows, D), _f32),
        scratch_types=scratch,
    )
    def perm_k(*refs):
        pos = 0
        table_hbm = refs[pos]; pos += 1
        ii_hbm = io_hbm = None
        if idx_in is not None:
            ii_hbm = refs[pos]; pos += 1
        if idx_out is not None:
            io_hbm = refs[pos]; pos += 1
        out_hbm = refs[pos]; pos += 1
        ii_v = io_v = None
        if idx_in is not None:
            ii_v = refs[pos:pos + NCK]; pos += NCK
        if idx_out is not None:
            io_v = refs[pos:pos + NCK]; pos += NCK
        rows_v = refs[pos:pos + NCK]; pos += NCK
        sem_g = refs[pos:pos + NCK]; pos += NCK
        sem_s = refs[pos:pos + NCK]

        wid = lax.axis_index("s") * NC + lax.axis_index("c")
        base = wid * npw

        gathers = []
        for k in range(NCK):
            if idx_in is not None:
                pltpu.sync_copy(ii_hbm.at[pl.ds(base + k * cn, cn)], ii_v[k])
            if idx_out is not None:
                pltpu.sync_copy(io_hbm.at[pl.ds(base + k * cn, cn)], io_v[k])
        for k in range(NCK):
            if idx_in is not None:
                g = pltpu.async_copy(table_hbm.at[ii_v[k]], rows_v[k],
                                     sem_g[k])
            else:
                g = pltpu.async_copy(table_hbm.at[pl.ds(base + k * cn, cn)],
                                     rows_v[k], sem_g[k])
            gathers.append(g)
        scatters = []
        for k in range(NCK):
            gathers[k].wait()
            if idx_out is not None:
                s = pltpu.async_copy(rows_v[k], out_hbm.at[io_v[k]], sem_s[k])
            else:
                s = pltpu.async_copy(rows_v[k],
                                     out_hbm.at[pl.ds(base + k * cn, cn)],
                                     sem_s[k])
            scatters.append(s)
        for s in scatters:
            s.wait()

    args = [table]
    if idx_in is not None:
        args.append(idx_in)
    if idx_out is not None:
        args.append(idx_out)
    return perm_k(*args)


def _embed_head(x, emb_W, emb_b, at_W1, at_b1, at_W2, at_b2):
    def body(x_ref, eW, eb, w1, b1, w2, b2, state_ref, lg_ref):
        st = jnp.dot(x_ref[...], eW[...], preferred_element_type=_f32) + eb[...]
        state_ref[...] = st
        h = jnp.maximum(jnp.dot(st, w1[0], preferred_element_type=_f32) + b1[0], 0.0)
        lg_ref[...] = jnp.dot(h, w2[0], preferred_element_type=_f32) + b2[0]

    return pl.pallas_call(
        body,
        grid=(B // BT,),
        in_specs=[
            pl.BlockSpec((BT, E), lambda i: (i, 0)),
            pl.BlockSpec((E, E), lambda i: (0, 0)),
            pl.BlockSpec((1, E), lambda i: (0, 0)),
            pl.BlockSpec((1, E, E), lambda i: (0, 0, 0)),
            pl.BlockSpec((1, 1, E), lambda i: (0, 0, 0)),
            pl.BlockSpec((1, E, NB), lambda i: (0, 0, 0)),
            pl.BlockSpec((1, 1, NB), lambda i: (0, 0, 0)),
        ],
        out_specs=[
            pl.BlockSpec((BT, E), lambda i: (i, 0)),
            pl.BlockSpec((BT, NB), lambda i: (i, 0)),
        ],
        out_shape=[
            jax.ShapeDtypeStruct((B, E), _f32),
            jax.ShapeDtypeStruct((B, NB), _f32),
        ],
        compiler_params=pltpu.CompilerParams(
            dimension_semantics=("arbitrary",)),
    )(x, emb_W, emb_b, at_W1, at_b1, at_W2, at_b2)


def _jump_with_addr(xs, te, nu, W1, b1, W2, b2, A1, Ab1, A2p, Ab2p):
    def body(te_r, nu_r, x_ref, w1, b1_, w2, b2_, a1, ab1, a2, ab2,
             ns_ref, lg_ref):
        t = pl.program_id(0)

        @pl.when(t < nu_r[0])
        def _():
            x = x_ref[...]
            norm = jnp.sqrt(jnp.sum(x * x, axis=1, keepdims=True))
            h1 = jnp.maximum(jnp.dot(x, w1[0], preferred_element_type=_f32) + b1_[0], 0.0)
            h2 = jnp.maximum(jnp.dot(h1, w2[0], preferred_element_type=_f32) + b2_[0], 0.0)
            ns = h2 / (norm + 1e-6)
            ns_ref[...] = ns
            av = jnp.maximum(jnp.dot(ns, a1[0], preferred_element_type=_f32) + ab1[0], 0.0)
            z = jnp.dot(av, a2[0], preferred_element_type=_f32) + ab2[0]
            lg_ref[...] = jnp.concatenate(
                [z, jnp.full((TB, 128 - NB), -1e9, _f32)], axis=1)

    def xmap(t, te_r, nu_r):
        return (jnp.minimum(t, nu_r[0] - 1), 0)

    def wmap(t, te_r, nu_r):
        return (te_r[t], 0, 0)

    grid_spec = pltpu.PrefetchScalarGridSpec(
        num_scalar_prefetch=2,
        grid=(NT,),
        in_specs=[
            pl.BlockSpec((TB, E), xmap),
            pl.BlockSpec((1, E, E), wmap),
            pl.BlockSpec((1, 1, E), wmap),
            pl.BlockSpec((1, E, E), wmap),
            pl.BlockSpec((1, 1, E), wmap),
            pl.BlockSpec((1, E, E), wmap),
            pl.BlockSpec((1, 1, E), wmap),
            pl.BlockSpec((1, E, NB), wmap),
            pl.BlockSpec((1, 1, NB), wmap),
        ],
        out_specs=[
            pl.BlockSpec((TB, E), xmap),
            pl.BlockSpec((TB, 128), xmap),
        ],
    )
    return pl.pallas_call(
        body,
        grid_spec=grid_spec,
        out_shape=[
            jax.ShapeDtypeStruct((PAD, E), _f32),
            jax.ShapeDtypeStruct((PAD, 128), _f32),
        ],
        compiler_params=pltpu.CompilerParams(
            dimension_semantics=("arbitrary",)),
    )(te, nu, xs, W1, b1, W2, b2, A1, Ab1, A2p, Ab2p)


def _jump_out(xs, te, nu, W1, b1, W2, b2, oW1, ob1, oW2, ob2):
    def body(te_r, nu_r, x_ref, w1, b1_, w2, b2_, ow1, ob1_, ow2, ob2_,
             o_ref):
        t = pl.program_id(0)

        @pl.when(t < nu_r[0])
        def _():
            x = x_ref[...]
            norm = jnp.sqrt(jnp.sum(x * x, axis=1, keepdims=True))
            h1 = jnp.maximum(jnp.dot(x, w1[0], preferred_element_type=_f32) + b1_[0], 0.0)
            h2 = jnp.maximum(jnp.dot(h1, w2[0], preferred_element_type=_f32) + b2_[0], 0.0)
            ns = h2 / (norm + 1e-6)
            ho = jnp.maximum(jnp.dot(ns, ow1[...], preferred_element_type=_f32) + ob1_[...], 0.0)
            o = jnp.dot(ho, ow2[...], preferred_element_type=_f32) + ob2_[...]
            o_ref[...] = jnp.concatenate(
                [o, jnp.zeros((TB, NCP - NCLASS), _f32)], axis=1)

    def xmap(t, te_r, nu_r):
        return (jnp.minimum(t, nu_r[0] - 1), 0)

    def wmap(t, te_r, nu_r):
        return (te_r[t], 0, 0)

    def cmap2(t, te_r, nu_r):
        return (0, 0)

    grid_spec = pltpu.PrefetchScalarGridSpec(
        num_scalar_prefetch=2,
        grid=(NT,),
        in_specs=[
            pl.BlockSpec((TB, E), xmap),
            pl.BlockSpec((1, E, E), wmap),
            pl.BlockSpec((1, 1, E), wmap),
            pl.BlockSpec((1, E, E), wmap),
            pl.BlockSpec((1, 1, E), wmap),
            pl.BlockSpec((E, E), cmap2),
            pl.BlockSpec((1, E), cmap2),
            pl.BlockSpec((E, NCLASS), cmap2),
            pl.BlockSpec((1, NCLASS), cmap2),
        ],
        out_specs=pl.BlockSpec((TB, NCP), xmap),
    )
    return pl.pallas_call(
        body,
        grid_spec=grid_spec,
        out_shape=jax.ShapeDtypeStruct((PAD, NCP), _f32),
        compiler_params=pltpu.CompilerParams(
            dimension_semantics=("arbitrary",)),
    )(te, nu, xs, W1, b1, W2, b2, oW1, ob1, oW2, ob2)


def _gumbel(i):
    with jax.ensure_compile_time_eval():
        key = jax.random.fold_in(jax.random.key(42), i)
        u = jax.random.uniform(key, (B, 1, NB), minval=1e-10, maxval=1.0)
        return (-jnp.log(-jnp.log(u))).reshape(B, NB)


def _route_build(lg, g):
    W = lg.shape[1]

    def body(lg_ref, g_ref, ppos_ref, te_ref, nu_ref):
        z = lg_ref[:, :NB] + g_ref[...]
        m = jnp.max(z, axis=1, keepdims=True)
        io8 = lax.broadcasted_iota(jnp.int32, (B, NB), 1)
        eidx = jnp.min(jnp.where(z == m, io8, NB), axis=1, keepdims=True)
        oh = (eidx == io8).astype(_f32)
        r = lax.broadcasted_iota(jnp.int32, (CS, CS), 0)
        c = lax.broadcasted_iota(jnp.int32, (CS, CS), 1)
        ltri = (c <= r).astype(_f32)
        parts = []
        carry = jnp.zeros((1, NB), _f32)
        for gi in range(B // CS):
            cs = jnp.dot(ltri, oh[gi * CS:(gi + 1) * CS, :],
                         preferred_element_type=_f32) + carry
            parts.append(cs)
            carry = cs[CS - 1:CS, :]
        csum = jnp.concatenate(parts, axis=0)
        rank = jnp.sum(oh * csum, axis=1, keepdims=True) - 1.0
        counts = carry
        tiles = jnp.floor((counts + (TB - 1)) * (1.0 / TB))
        u8r = lax.broadcasted_iota(jnp.int32, (NB, NB), 0)
        u8c = lax.broadcasted_iota(jnp.int32, (NB, NB), 1)
        utri = (u8r <= u8c).astype(_f32)
        cumt = jnp.dot(tiles, utri, preferred_element_type=_f32)
        nused = cumt[0:1, NB - 1:NB]
        tile_start = (cumt - tiles) * float(TB)
        ppos = jnp.sum(oh * tile_start, axis=1, keepdims=True) + rank
        ppos_ref[...] = ppos.astype(jnp.int32)
        tio = lax.broadcasted_iota(jnp.int32, (NT, 1), 0).astype(_f32)
        tcl = jnp.minimum(tio, nused - 1.0)
        te_ref[...] = jnp.sum((cumt <= tcl).astype(jnp.int32), axis=1,
                              keepdims=True)
        nu_ref[...] = nused.astype(jnp.int32)

    ppos, te, nu = pl.pallas_call(
        body,
        in_specs=[
            pl.BlockSpec((B, W), lambda: (0, 0)),
            pl.BlockSpec((B, NB), lambda: (0, 0)),
        ],
        out_specs=[
            pl.BlockSpec((B, 1), lambda: (0, 0)),
            pl.BlockSpec((NT, 1), lambda: (0, 0)),
            pl.BlockSpec((1, 1), lambda: (0, 0)),
        ],
        out_shape=[
            jax.ShapeDtypeStruct((B, 1), jnp.int32),
            jax.ShapeDtypeStruct((NT, 1), jnp.int32),
            jax.ShapeDtypeStruct((1, 1), jnp.int32),
        ],
    )(lg, g)
    return ppos.reshape(B), te.reshape(NT), nu.reshape(1)


def kernel(x, emb_W, emb_b, st_W1, st_b1, st_W2, st_b2,
           at_W1, at_b1, at_W2, at_b2, out_W1, out_b1, out_W2, out_b2):
    emb_b2 = emb_b.reshape(1, E)
    st_b1r = st_b1.reshape(NB, 1, E)
    st_b2r = st_b2.reshape(NB, 1, E)
    at_b1r = at_b1.reshape(NB, 1, E)
    at_b2r = at_b2.reshape(NB, 1, NB)

    at_b2r = at_b2.reshape(NB, 1, NB)

    state, lg0 = _embed_head(x, emb_W, emb_b2, at_W1, at_b1r, at_W2, at_b2r)

    ppos1, te1, nu1 = _route_build(lg0, _gumbel(0))
    xs1 = _sc_perm(state, None, ppos1, PAD)
    ns1, lgp1 = _jump_with_addr(xs1, te1, nu1, st_W1, st_b1r, st_W2, st_b2r,
                                at_W1, at_b1r, at_W2, at_b2r)

    lg1 = _sc_perm(lgp1, ppos1, None, B)
    ppos2, te2, nu2 = _route_build(lg1, _gumbel(1))
    xs2 = _sc_perm(ns1, ppos1, ppos2, PAD)
    o_sorted = _jump_out(xs2, te2, nu2, st_W1, st_b1r, st_W2, st_b2r,
                         out_W1, out_b1.reshape(1, E),
                         out_W2, out_b2.reshape(1, NCLASS))

    o = _sc_perm(o_sorted, ppos2, None, B)
    return o[:, :NCLASS]


# speedup vs baseline: 1.0148x; 1.0071x over previous
import functools

import jax
import jax.numpy as jnp
from jax import lax
from jax.experimental import pallas as pl
from jax.experimental.pallas import tpu as pltpu
from jax.experimental.pallas import tpu_sc as plsc

B = 2048
E = 1024
NB = 8
NCLASS = 1000
TB = 128
NT = 24
PAD = NT * TB
BT = 256
CS = 128
NCP = 1024

_f32 = jnp.float32


def _sc_perm(table, idx_in, idx_out, out_rows):
    V, D = table.shape
    Bc = (idx_in if idx_in is not None else idx_out).shape[0]
    info = plsc.get_sparse_core_info()
    NC, NS = info.num_cores, info.num_subcores
    NW = NC * NS
    npw = Bc // NW
    NCK = 1
    cn = npw // NCK
    assert Bc % (8 * NW) == 0 and D % 128 == 0 and cn % 8 == 0
    mesh = plsc.VectorSubcoreMesh(core_axis_name="c", subcore_axis_name="s")

    scratch = []
    if idx_in is not None:
        scratch += [pltpu.VMEM((cn,), jnp.int32)] * NCK
    if idx_out is not None:
        scratch += [pltpu.VMEM((cn,), jnp.int32)] * NCK
    scratch += [pltpu.VMEM((cn, D), _f32)] * NCK
    scratch += [pltpu.SemaphoreType.DMA] * (2 * NCK)

    @functools.partial(
        pl.kernel, mesh=mesh,
        out_type=jax.ShapeDtypeStruct((out_rows, D), _f32),
        scratch_types=scratch,
    )
    def perm_k(*refs):
        pos = 0
        table_hbm = refs[pos]; pos += 1
        ii_hbm = io_hbm = None
        if idx_in is not None:
            ii_hbm = refs[pos]; pos += 1
        if idx_out is not None:
            io_hbm = refs[pos]; pos += 1
        out_hbm = refs[pos]; pos += 1
        ii_v = io_v = None
        if idx_in is not None:
            ii_v = refs[pos:pos + NCK]; pos += NCK
        if idx_out is not None:
            io_v = refs[pos:pos + NCK]; pos += NCK
        rows_v = refs[pos:pos + NCK]; pos += NCK
        sem_g = refs[pos:pos + NCK]; pos += NCK
        sem_s = refs[pos:pos + NCK]

        wid = lax.axis_index("s") * NC + lax.axis_index("c")
        base = wid * npw

        gathers = []
        for k in range(NCK):
            if idx_in is not None:
                pltpu.sync_copy(ii_hbm.at[pl.ds(base + k * cn, cn)], ii_v[k])
            if idx_out is not None:
                pltpu.sync_copy(io_hbm.at[pl.ds(base + k * cn, cn)], io_v[k])
        for k in range(NCK):
            if idx_in is not None:
                g = pltpu.async_copy(table_hbm.at[ii_v[k]], rows_v[k],
                                     sem_g[k])
            else:
                g = pltpu.async_copy(table_hbm.at[pl.ds(base + k * cn, cn)],
                                     rows_v[k], sem_g[k])
            gathers.append(g)
        scatters = []
        for k in range(NCK):
            gathers[k].wait()
            if idx_out is not None:
                s = pltpu.async_copy(rows_v[k], out_hbm.at[io_v[k]], sem_s[k])
            else:
                s = pltpu.async_copy(rows_v[k],
                                     out_hbm.at[pl.ds(base + k * cn, cn)],
                                     sem_s[k])
            scatters.append(s)
        for s in scatters:
            s.wait()

    args = [table]
    if idx_in is not None:
        args.append(idx_in)
    if idx_out is not None:
        args.append(idx_out)
    return perm_k(*args)


def _embed_head(x, emb_W, emb_b, at_W1, at_b1, at_W2, at_b2, g):
    def body(x_ref, eW, eb, w1, b1, w2, b2, g_ref,
             state_ref, ppos_ref, te_ref, nu_ref, z_acc):
        i = pl.program_id(0)
        st = jnp.dot(x_ref[...], eW[...], preferred_element_type=_f32) + eb[...]
        state_ref[...] = st
        h = jnp.maximum(jnp.dot(st, w1[0], preferred_element_type=_f32) + b1[0], 0.0)
        z_acc[pl.ds(i * BT, BT), :] = (
            jnp.dot(h, w2[0], preferred_element_type=_f32) + b2[0]
            + g_ref[...])

        @pl.when(i == B // BT - 1)
        def _():
            _routing_math(z_acc[...], ppos_ref, te_ref, nu_ref)

    state, ppos, te, nu = pl.pallas_call(
        body,
        grid=(B // BT,),
        in_specs=[
            pl.BlockSpec((BT, E), lambda i: (i, 0)),
            pl.BlockSpec((E, E), lambda i: (0, 0)),
            pl.BlockSpec((1, E), lambda i: (0, 0)),
            pl.BlockSpec((1, E, E), lambda i: (0, 0, 0)),
            pl.BlockSpec((1, 1, E), lambda i: (0, 0, 0)),
            pl.BlockSpec((1, E, NB), lambda i: (0, 0, 0)),
            pl.BlockSpec((1, 1, NB), lambda i: (0, 0, 0)),
            pl.BlockSpec((BT, NB), lambda i: (i, 0)),
        ],
        out_specs=[
            pl.BlockSpec((BT, E), lambda i: (i, 0)),
            pl.BlockSpec((B, 1), lambda i: (0, 0)),
            pl.BlockSpec((NT, 1), lambda i: (0, 0)),
            pl.BlockSpec((1, 1), lambda i: (0, 0)),
        ],
        out_shape=[
            jax.ShapeDtypeStruct((B, E), _f32),
            jax.ShapeDtypeStruct((B, 1), jnp.int32),
            jax.ShapeDtypeStruct((NT, 1), jnp.int32),
            jax.ShapeDtypeStruct((1, 1), jnp.int32),
        ],
        scratch_shapes=[pltpu.VMEM((B, NB), _f32)],
        compiler_params=pltpu.CompilerParams(
            dimension_semantics=("arbitrary",)),
    )(x, emb_W, emb_b, at_W1, at_b1, at_W2, at_b2, g)
    return state, ppos.reshape(B), te.reshape(NT), nu.reshape(1)


def _jump_with_addr(xs, te, nu, W1, b1, W2, b2, A1, Ab1, A2p, Ab2p):
    def body(te_r, nu_r, x_ref, w1, b1_, w2, b2_, a1, ab1, a2, ab2,
             ns_ref, lg_ref):
        t = pl.program_id(0)

        @pl.when(t < nu_r[0])
        def _():
            x = x_ref[...]
            norm = jnp.sqrt(jnp.sum(x * x, axis=1, keepdims=True))
            h1 = jnp.maximum(jnp.dot(x, w1[0], preferred_element_type=_f32) + b1_[0], 0.0)
            h2 = jnp.maximum(jnp.dot(h1, w2[0], preferred_element_type=_f32) + b2_[0], 0.0)
            ns = h2 / (norm + 1e-6)
            ns_ref[...] = ns
            av = jnp.maximum(jnp.dot(ns, a1[0], preferred_element_type=_f32) + ab1[0], 0.0)
            z = jnp.dot(av, a2[0], preferred_element_type=_f32) + ab2[0]
            lg_ref[...] = jnp.concatenate(
                [z, jnp.full((TB, 128 - NB), -1e9, _f32)], axis=1)

    def xmap(t, te_r, nu_r):
        return (jnp.minimum(t, nu_r[0] - 1), 0)

    def wmap(t, te_r, nu_r):
        return (te_r[t], 0, 0)

    grid_spec = pltpu.PrefetchScalarGridSpec(
        num_scalar_prefetch=2,
        grid=(NT,),
        in_specs=[
            pl.BlockSpec((TB, E), xmap),
            pl.BlockSpec((1, E, E), wmap),
            pl.BlockSpec((1, 1, E), wmap),
            pl.BlockSpec((1, E, E), wmap),
            pl.BlockSpec((1, 1, E), wmap),
            pl.BlockSpec((1, E, E), wmap),
            pl.BlockSpec((1, 1, E), wmap),
            pl.BlockSpec((1, E, NB), wmap),
            pl.BlockSpec((1, 1, NB), wmap),
        ],
        out_specs=[
            pl.BlockSpec((TB, E), xmap),
            pl.BlockSpec((TB, 128), xmap),
        ],
    )
    return pl.pallas_call(
        body,
        grid_spec=grid_spec,
        out_shape=[
            jax.ShapeDtypeStruct((PAD, E), _f32),
            jax.ShapeDtypeStruct((PAD, 128), _f32),
        ],
        compiler_params=pltpu.CompilerParams(
            dimension_semantics=("arbitrary",)),
    )(te, nu, xs, W1, b1, W2, b2, A1, Ab1, A2p, Ab2p)


def _jump_out(xs, te, nu, W1, b1, W2, b2, oW1, ob1, oW2, ob2):
    def body(te_r, nu_r, x_ref, w1, b1_, w2, b2_, ow1, ob1_, ow2, ob2_,
             o_ref):
        t = pl.program_id(0)

        @pl.when(t < nu_r[0])
        def _():
            x = x_ref[...]
            norm = jnp.sqrt(jnp.sum(x * x, axis=1, keepdims=True))
            h1 = jnp.maximum(jnp.dot(x, w1[0], preferred_element_type=_f32) + b1_[0], 0.0)
            h2 = jnp.maximum(jnp.dot(h1, w2[0], preferred_element_type=_f32) + b2_[0], 0.0)
            ns = h2 / (norm + 1e-6)
            ho = jnp.maximum(jnp.dot(ns, ow1[...], preferred_element_type=_f32) + ob1_[...], 0.0)
            o = jnp.dot(ho, ow2[...], preferred_element_type=_f32) + ob2_[...]
            o_ref[...] = jnp.concatenate(
                [o, jnp.zeros((TB, NCP - NCLASS), _f32)], axis=1)

    def xmap(t, te_r, nu_r):
        return (jnp.minimum(t, nu_r[0] - 1), 0)

    def wmap(t, te_r, nu_r):
        return (te_r[t], 0, 0)

    def cmap2(t, te_r, nu_r):
        return (0, 0)

    grid_spec = pltpu.PrefetchScalarGridSpec(
        num_scalar_prefetch=2,
        grid=(NT,),
        in_specs=[
            pl.BlockSpec((TB, E), xmap),
            pl.BlockSpec((1, E, E), wmap),
            pl.BlockSpec((1, 1, E), wmap),
            pl.BlockSpec((1, E, E), wmap),
            pl.BlockSpec((1, 1, E), wmap),
            pl.BlockSpec((E, E), cmap2),
            pl.BlockSpec((1, E), cmap2),
            pl.BlockSpec((E, NCLASS), cmap2),
            pl.BlockSpec((1, NCLASS), cmap2),
        ],
        out_specs=pl.BlockSpec((TB, NCP), xmap),
    )
    return pl.pallas_call(
        body,
        grid_spec=grid_spec,
        out_shape=jax.ShapeDtypeStruct((PAD, NCP), _f32),
        compiler_params=pltpu.CompilerParams(
            dimension_semantics=("arbitrary",)),
    )(te, nu, xs, W1, b1, W2, b2, oW1, ob1, oW2, ob2)


def _gumbel(i):
    with jax.ensure_compile_time_eval():
        key = jax.random.fold_in(jax.random.key(42), i)
        u = jax.random.uniform(key, (B, 1, NB), minval=1e-10, maxval=1.0)
        return (-jnp.log(-jnp.log(u))).reshape(B, NB)


def _routing_math(z, ppos_ref, te_ref, nu_ref):
    m = jnp.max(z, axis=1, keepdims=True)
    io8 = lax.broadcasted_iota(jnp.int32, (B, NB), 1)
    eidx = jnp.min(jnp.where(z == m, io8, NB), axis=1, keepdims=True)
    oh = (eidx == io8).astype(_f32)
    r = lax.broadcasted_iota(jnp.int32, (CS, CS), 0)
    c = lax.broadcasted_iota(jnp.int32, (CS, CS), 1)
    ltri = (c <= r).astype(_f32)
    parts = []
    carry = jnp.zeros((1, NB), _f32)
    for gi in range(B // CS):
        cs = jnp.dot(ltri, oh[gi * CS:(gi + 1) * CS, :],
                     preferred_element_type=_f32) + carry
        parts.append(cs)
        carry = cs[CS - 1:CS, :]
    csum = jnp.concatenate(parts, axis=0)
    rank = jnp.sum(oh * csum, axis=1, keepdims=True) - 1.0
    counts = carry
    tiles = jnp.floor((counts + (TB - 1)) * (1.0 / TB))
    u8r = lax.broadcasted_iota(jnp.int32, (NB, NB), 0)
    u8c = lax.broadcasted_iota(jnp.int32, (NB, NB), 1)
    utri = (u8r <= u8c).astype(_f32)
    cumt = jnp.dot(tiles, utri, preferred_element_type=_f32)
    nused = cumt[0:1, NB - 1:NB]
    tile_start = (cumt - tiles) * float(TB)
    ppos = jnp.sum(oh * tile_start, axis=1, keepdims=True) + rank
    ppos_ref[...] = ppos.astype(jnp.int32)
    tio = lax.broadcasted_iota(jnp.int32, (NT, 1), 0).astype(_f32)
    tcl = jnp.minimum(tio, nused - 1.0)
    te_ref[...] = jnp.sum((cumt <= tcl).astype(jnp.int32), axis=1,
                          keepdims=True)
    nu_ref[...] = nused.astype(jnp.int32)


def _route_build(lg, g):
    W = lg.shape[1]

    def body(lg_ref, g_ref, ppos_ref, te_ref, nu_ref):
        z = lg_ref[:, :NB] + g_ref[...]
        _routing_math(z, ppos_ref, te_ref, nu_ref)

    ppos, te, nu = pl.pallas_call(
        body,
        in_specs=[
            pl.BlockSpec((B, W), lambda: (0, 0)),
            pl.BlockSpec((B, NB), lambda: (0, 0)),
        ],
        out_specs=[
            pl.BlockSpec((B, 1), lambda: (0, 0)),
            pl.BlockSpec((NT, 1), lambda: (0, 0)),
            pl.BlockSpec((1, 1), lambda: (0, 0)),
        ],
        out_shape=[
            jax.ShapeDtypeStruct((B, 1), jnp.int32),
            jax.ShapeDtypeStruct((NT, 1), jnp.int32),
            jax.ShapeDtypeStruct((1, 1), jnp.int32),
        ],
    )(lg, g)
    return ppos.reshape(B), te.reshape(NT), nu.reshape(1)


def kernel(x, emb_W, emb_b, st_W1, st_b1, st_W2, st_b2,
           at_W1, at_b1, at_W2, at_b2, out_W1, out_b1, out_W2, out_b2):
    emb_b2 = emb_b.reshape(1, E)
    st_b1r = st_b1.reshape(NB, 1, E)
    st_b2r = st_b2.reshape(NB, 1, E)
    at_b1r = at_b1.reshape(NB, 1, E)
    at_b2r = at_b2.reshape(NB, 1, NB)

    at_b2r = at_b2.reshape(NB, 1, NB)

    state, ppos1, te1, nu1 = _embed_head(x, emb_W, emb_b2, at_W1, at_b1r,
                                         at_W2, at_b2r, _gumbel(0))

    xs1 = _sc_perm(state, None, ppos1, PAD)
    ns1, lgp1 = _jump_with_addr(xs1, te1, nu1, st_W1, st_b1r, st_W2, st_b2r,
                                at_W1, at_b1r, at_W2, at_b2r)

    lg1 = _sc_perm(lgp1, ppos1, None, B)
    ppos2, te2, nu2 = _route_build(lg1, _gumbel(1))
    xs2 = _sc_perm(ns1, ppos1, ppos2, PAD)
    o_sorted = _jump_out(xs2, te2, nu2, st_W1, st_b1r, st_W2, st_b2r,
                         out_W1, out_b1.reshape(1, E),
                         out_W2, out_b2.reshape(1, NCLASS))

    o = _sc_perm(o_sorted, ppos2, None, B)
    return o[:, :NCLASS]


# speedup vs baseline: 1.0353x; 1.0202x over previous
import functools

import jax
import jax.numpy as jnp
from jax import lax
from jax.experimental import pallas as pl
from jax.experimental.pallas import tpu as pltpu
from jax.experimental.pallas import tpu_sc as plsc

B = 2048
E = 1024
NB = 8
NCLASS = 1000
TB = 128
NT = 24
PAD = NT * TB
BT = 256
CS = 128
NCP = 1024

_f32 = jnp.float32


def _sc_perm(table, idx_in, idx_out, out_rows):
    V, D = table.shape
    Bc = (idx_in if idx_in is not None else idx_out).shape[0]
    info = plsc.get_sparse_core_info()
    NC, NS = info.num_cores, info.num_subcores
    NW = NC * NS
    npw = Bc // NW
    NCK = 1
    cn = npw // NCK
    assert Bc % (8 * NW) == 0 and D % 128 == 0 and cn % 8 == 0
    mesh = plsc.VectorSubcoreMesh(core_axis_name="c", subcore_axis_name="s")

    scratch = []
    if idx_in is not None:
        scratch += [pltpu.VMEM((cn,), jnp.int32)] * NCK
    if idx_out is not None:
        scratch += [pltpu.VMEM((cn,), jnp.int32)] * NCK
    scratch += [pltpu.VMEM((cn, D), _f32)] * NCK
    scratch += [pltpu.SemaphoreType.DMA] * (2 * NCK)

    @functools.partial(
        pl.kernel, mesh=mesh,
        out_type=jax.ShapeDtypeStruct((out_rows, D), _f32),
        scratch_types=scratch,
    )
    def perm_k(*refs):
        pos = 0
        table_hbm = refs[pos]; pos += 1
        ii_hbm = io_hbm = None
        if idx_in is not None:
            ii_hbm = refs[pos]; pos += 1
        if idx_out is not None:
            io_hbm = refs[pos]; pos += 1
        out_hbm = refs[pos]; pos += 1
        ii_v = io_v = None
        if idx_in is not None:
            ii_v = refs[pos:pos + NCK]; pos += NCK
        if idx_out is not None:
            io_v = refs[pos:pos + NCK]; pos += NCK
        rows_v = refs[pos:pos + NCK]; pos += NCK
        sem_g = refs[pos:pos + NCK]; pos += NCK
        sem_s = refs[pos:pos + NCK]

        wid = lax.axis_index("s") * NC + lax.axis_index("c")
        base = wid * npw

        gathers = []
        for k in range(NCK):
            if idx_in is not None:
                pltpu.sync_copy(ii_hbm.at[pl.ds(base + k * cn, cn)], ii_v[k])
            if idx_out is not None:
                pltpu.sync_copy(io_hbm.at[pl.ds(base + k * cn, cn)], io_v[k])
        for k in range(NCK):
            if idx_in is not None:
                g = pltpu.async_copy(table_hbm.at[ii_v[k]], rows_v[k],
                                     sem_g[k])
            else:
                g = pltpu.async_copy(table_hbm.at[pl.ds(base + k * cn, cn)],
                                     rows_v[k], sem_g[k])
            gathers.append(g)
        scatters = []
        for k in range(NCK):
            gathers[k].wait()
            if idx_out is not None:
                s = pltpu.async_copy(rows_v[k], out_hbm.at[io_v[k]], sem_s[k])
            else:
                s = pltpu.async_copy(rows_v[k],
                                     out_hbm.at[pl.ds(base + k * cn, cn)],
                                     sem_s[k])
            scatters.append(s)
        for s in scatters:
            s.wait()

    args = [table]
    if idx_in is not None:
        args.append(idx_in)
    if idx_out is not None:
        args.append(idx_out)
    return perm_k(*args)


def _embed_head(x, emb_W, emb_b, at_W1, at_b1, at_W2, at_b2, g):
    def body(x_ref, eW, eb, w1, b1, w2, b2, g_ref,
             state_ref, ppos_ref, te_ref, nu_ref, z_acc):
        i = pl.program_id(0)
        st = jnp.dot(x_ref[...], eW[...], preferred_element_type=_f32) + eb[...]
        state_ref[...] = st
        h = jnp.maximum(jnp.dot(st, w1[0], preferred_element_type=_f32) + b1[0], 0.0)
        z_acc[pl.ds(i * BT, BT), :] = (
            jnp.dot(h, w2[0], preferred_element_type=_f32) + b2[0]
            + g_ref[...])

        @pl.when(i == B // BT - 1)
        def _():
            _routing_math(z_acc[...], ppos_ref, te_ref, nu_ref)

    state, ppos, te, nu = pl.pallas_call(
        body,
        grid=(B // BT,),
        in_specs=[
            pl.BlockSpec((BT, E), lambda i: (i, 0)),
            pl.BlockSpec((E, E), lambda i: (0, 0)),
            pl.BlockSpec((1, E), lambda i: (0, 0)),
            pl.BlockSpec((1, E, E), lambda i: (0, 0, 0)),
            pl.BlockSpec((1, 1, E), lambda i: (0, 0, 0)),
            pl.BlockSpec((1, E, NB), lambda i: (0, 0, 0)),
            pl.BlockSpec((1, 1, NB), lambda i: (0, 0, 0)),
            pl.BlockSpec((BT, NB), lambda i: (i, 0)),
        ],
        out_specs=[
            pl.BlockSpec((BT, E), lambda i: (i, 0)),
            pl.BlockSpec((B,), lambda i: (0,)),
            pl.BlockSpec((NT,), lambda i: (0,)),
            pl.BlockSpec((1,), lambda i: (0,)),
        ],
        out_shape=[
            jax.ShapeDtypeStruct((B, E), _f32),
            jax.ShapeDtypeStruct((B,), jnp.int32),
            jax.ShapeDtypeStruct((NT,), jnp.int32),
            jax.ShapeDtypeStruct((1,), jnp.int32),
        ],
        scratch_shapes=[pltpu.VMEM((B, NB), _f32)],
        compiler_params=pltpu.CompilerParams(
            dimension_semantics=("arbitrary",)),
    )(x, emb_W, emb_b, at_W1, at_b1, at_W2, at_b2, g)
    return state, ppos, te, nu


def _jump_with_addr(xs, te, nu, W1, b1, W2, b2, A1, Ab1, A2p, Ab2p):
    def body(te_r, nu_r, x_ref, w1, b1_, w2, b2_, a1, ab1, a2, ab2,
             ns_ref, lg_ref):
        t = pl.program_id(0)

        @pl.when(t < nu_r[0])
        def _():
            x = x_ref[...]
            norm = jnp.sqrt(jnp.sum(x * x, axis=1, keepdims=True))
            h1 = jnp.maximum(jnp.dot(x, w1[0], preferred_element_type=_f32) + b1_[0], 0.0)
            h2 = jnp.maximum(jnp.dot(h1, w2[0], preferred_element_type=_f32) + b2_[0], 0.0)
            ns = h2 / (norm + 1e-6)
            ns_ref[...] = ns
            av = jnp.maximum(jnp.dot(ns, a1[0], preferred_element_type=_f32) + ab1[0], 0.0)
            z = jnp.dot(av, a2[0], preferred_element_type=_f32) + ab2[0]
            lg_ref[...] = jnp.concatenate(
                [z, jnp.full((TB, 128 - NB), -1e9, _f32)], axis=1)

    def xmap(t, te_r, nu_r):
        return (jnp.minimum(t, nu_r[0] - 1), 0)

    def wmap(t, te_r, nu_r):
        return (te_r[t], 0, 0)

    grid_spec = pltpu.PrefetchScalarGridSpec(
        num_scalar_prefetch=2,
        grid=(NT,),
        in_specs=[
            pl.BlockSpec((TB, E), xmap),
            pl.BlockSpec((1, E, E), wmap),
            pl.BlockSpec((1, 1, E), wmap),
            pl.BlockSpec((1, E, E), wmap),
            pl.BlockSpec((1, 1, E), wmap),
            pl.BlockSpec((1, E, E), wmap),
            pl.BlockSpec((1, 1, E), wmap),
            pl.BlockSpec((1, E, NB), wmap),
            pl.BlockSpec((1, 1, NB), wmap),
        ],
        out_specs=[
            pl.BlockSpec((TB, E), xmap),
            pl.BlockSpec((TB, 128), xmap),
        ],
    )
    return pl.pallas_call(
        body,
        grid_spec=grid_spec,
        out_shape=[
            jax.ShapeDtypeStruct((PAD, E), _f32),
            jax.ShapeDtypeStruct((PAD, 128), _f32),
        ],
        compiler_params=pltpu.CompilerParams(
            dimension_semantics=("arbitrary",)),
    )(te, nu, xs, W1, b1, W2, b2, A1, Ab1, A2p, Ab2p)


def _jump_out(xs, te, nu, W1, b1, W2, b2, oW1, ob1, oW2, ob2):
    def body(te_r, nu_r, x_ref, w1, b1_, w2, b2_, ow1, ob1_, ow2, ob2_,
             o_ref):
        t = pl.program_id(0)

        @pl.when(t < nu_r[0])
        def _():
            x = x_ref[...]
            norm = jnp.sqrt(jnp.sum(x * x, axis=1, keepdims=True))
            h1 = jnp.maximum(jnp.dot(x, w1[0], preferred_element_type=_f32) + b1_[0], 0.0)
            h2 = jnp.maximum(jnp.dot(h1, w2[0], preferred_element_type=_f32) + b2_[0], 0.0)
            ns = h2 / (norm + 1e-6)
            ho = jnp.maximum(jnp.dot(ns, ow1[...], preferred_element_type=_f32) + ob1_[...], 0.0)
            o = jnp.dot(ho, ow2[...], preferred_element_type=_f32) + ob2_[...]
            o_ref[...] = jnp.concatenate(
                [o, jnp.zeros((TB, NCP - NCLASS), _f32)], axis=1)

    def xmap(t, te_r, nu_r):
        return (jnp.minimum(t, nu_r[0] - 1), 0)

    def wmap(t, te_r, nu_r):
        return (te_r[t], 0, 0)

    def cmap2(t, te_r, nu_r):
        return (0, 0)

    grid_spec = pltpu.PrefetchScalarGridSpec(
        num_scalar_prefetch=2,
        grid=(NT,),
        in_specs=[
            pl.BlockSpec((TB, E), xmap),
            pl.BlockSpec((1, E, E), wmap),
            pl.BlockSpec((1, 1, E), wmap),
            pl.BlockSpec((1, E, E), wmap),
            pl.BlockSpec((1, 1, E), wmap),
            pl.BlockSpec((E, E), cmap2),
            pl.BlockSpec((1, E), cmap2),
            pl.BlockSpec((E, NCLASS), cmap2),
            pl.BlockSpec((1, NCLASS), cmap2),
        ],
        out_specs=pl.BlockSpec((TB, NCP), xmap),
    )
    return pl.pallas_call(
        body,
        grid_spec=grid_spec,
        out_shape=jax.ShapeDtypeStruct((PAD, NCP), _f32),
        compiler_params=pltpu.CompilerParams(
            dimension_semantics=("arbitrary",)),
    )(te, nu, xs, W1, b1, W2, b2, oW1, ob1, oW2, ob2)


def _gumbel(i):
    with jax.ensure_compile_time_eval():
        key = jax.random.fold_in(jax.random.key(42), i)
        u = jax.random.uniform(key, (B, 1, NB), minval=1e-10, maxval=1.0)
        return (-jnp.log(-jnp.log(u))).reshape(B, NB)


def _routing_math(z, ppos_ref, te_ref, nu_ref):
    m = jnp.max(z, axis=1, keepdims=True)
    io8 = lax.broadcasted_iota(jnp.int32, (B, NB), 1)
    eidx = jnp.min(jnp.where(z == m, io8, NB), axis=1, keepdims=True)
    oh = (eidx == io8).astype(_f32)
    r = lax.broadcasted_iota(jnp.int32, (CS, CS), 0)
    c = lax.broadcasted_iota(jnp.int32, (CS, CS), 1)
    ltri = (c <= r).astype(_f32)
    parts = []
    carry = jnp.zeros((1, NB), _f32)
    for gi in range(B // CS):
        cs = jnp.dot(ltri, oh[gi * CS:(gi + 1) * CS, :],
                     preferred_element_type=_f32) + carry
        parts.append(cs)
        carry = cs[CS - 1:CS, :]
    csum = jnp.concatenate(parts, axis=0)
    rank = jnp.sum(oh * csum, axis=1, keepdims=True) - 1.0
    counts = carry
    tiles = jnp.floor((counts + (TB - 1)) * (1.0 / TB))
    u8r = lax.broadcasted_iota(jnp.int32, (NB, NB), 0)
    u8c = lax.broadcasted_iota(jnp.int32, (NB, NB), 1)
    utri = (u8r <= u8c).astype(_f32)
    cumt = jnp.dot(tiles, utri, preferred_element_type=_f32)
    nused = cumt[0:1, NB - 1:NB]
    tile_start = (cumt - tiles) * float(TB)
    ppos = jnp.sum(oh * tile_start, axis=1, keepdims=True) + rank
    ppos_ref[...] = ppos.astype(jnp.int32).reshape(B)
    tio = lax.broadcasted_iota(jnp.int32, (NT, 1), 0).astype(_f32)
    tcl = jnp.minimum(tio, nused - 1.0)
    te_ref[...] = jnp.sum((cumt <= tcl).astype(jnp.int32), axis=1,
                          keepdims=True).reshape(NT)
    nu_ref[...] = nused.astype(jnp.int32).reshape(1)


def _route_build(lg, g):
    W = lg.shape[1]

    def body(lg_ref, g_ref, ppos_ref, te_ref, nu_ref):
        z = lg_ref[:, :NB] + g_ref[...]
        _routing_math(z, ppos_ref, te_ref, nu_ref)

    ppos, te, nu = pl.pallas_call(
        body,
        in_specs=[
            pl.BlockSpec((B, W), lambda: (0, 0)),
            pl.BlockSpec((B, NB), lambda: (0, 0)),
        ],
        out_specs=[
            pl.BlockSpec((B,), lambda: (0,)),
            pl.BlockSpec((NT,), lambda: (0,)),
            pl.BlockSpec((1,), lambda: (0,)),
        ],
        out_shape=[
            jax.ShapeDtypeStruct((B,), jnp.int32),
            jax.ShapeDtypeStruct((NT,), jnp.int32),
            jax.ShapeDtypeStruct((1,), jnp.int32),
        ],
    )(lg, g)
    return ppos, te, nu


def kernel(x, emb_W, emb_b, st_W1, st_b1, st_W2, st_b2,
           at_W1, at_b1, at_W2, at_b2, out_W1, out_b1, out_W2, out_b2):
    emb_b2 = emb_b.reshape(1, E)
    st_b1r = st_b1.reshape(NB, 1, E)
    st_b2r = st_b2.reshape(NB, 1, E)
    at_b1r = at_b1.reshape(NB, 1, E)
    at_b2r = at_b2.reshape(NB, 1, NB)

    at_b2r = at_b2.reshape(NB, 1, NB)

    state, ppos1, te1, nu1 = _embed_head(x, emb_W, emb_b2, at_W1, at_b1r,
                                         at_W2, at_b2r, _gumbel(0))

    xs1 = _sc_perm(state, None, ppos1, PAD)
    ns1, lgp1 = _jump_with_addr(xs1, te1, nu1, st_W1, st_b1r, st_W2, st_b2r,
                                at_W1, at_b1r, at_W2, at_b2r)

    lg1 = _sc_perm(lgp1, ppos1, None, B)
    ppos2, te2, nu2 = _route_build(lg1, _gumbel(1))
    xs2 = _sc_perm(ns1, ppos1, ppos2, PAD)
    o_sorted = _jump_out(xs2, te2, nu2, st_W1, st_b1r, st_W2, st_b2r,
                         out_W1, out_b1.reshape(1, E),
                         out_W2, out_b2.reshape(1, NCLASS))

    o = _sc_perm(o_sorted, ppos2, None, B)
    return o[:, :NCLASS]
